# baseline (speedup 1.0000x reference)
import jax
import jax.numpy as jnp
from jax import lax
from jax.experimental import pallas as pl
from jax.experimental.pallas import tpu as pltpu
from jax.experimental.pallas import tpu_sc as plsc

N = 10000
E = 320000
H = 8
D = 16
F = 128
HH = 4
FH = 64
FW = 80

NC = 2
NS = 16
EPS = E // NS
G = 400
NCHUNK = EPS // G

ROWS_PER_SUB = 624
TAIL_ROW0 = ROWS_PER_SUB * NS
TAIL = N - TAIL_ROW0


def _front_body(x_ref, w_ref, al_ref, ar_ref, fa_ref, fb_ref, r_ref):
    f = jnp.dot(x_ref[...], w_ref[...], preferred_element_type=jnp.float32)
    ltab = jnp.dot(f, al_ref[...], preferred_element_type=jnp.float32)
    fa_ref[...] = jnp.concatenate([f[:, :FH], ltab], axis=1)
    fb_ref[...] = jnp.concatenate([f[:, FH:], ltab], axis=1)
    r_ref[...] = jnp.dot(f, ar_ref[...], preferred_element_type=jnp.float32)


def _front(x, W, al16, ar16):
    blk = 1000
    return pl.pallas_call(
        _front_body,
        grid=(N // blk,),
        in_specs=[
            pl.BlockSpec((blk, F), lambda i: (i, 0)),
            pl.BlockSpec((F, F), lambda i: (0, 0)),
            pl.BlockSpec((F, D), lambda i: (0, 0)),
            pl.BlockSpec((F, D), lambda i: (0, 0)),
        ],
        out_specs=[
            pl.BlockSpec((blk, FW), lambda i: (i, 0)),
            pl.BlockSpec((blk, FW), lambda i: (i, 0)),
            pl.BlockSpec((blk, D), lambda i: (i, 0)),
        ],
        out_shape=[
            jax.ShapeDtypeStruct((N, FW), jnp.float32),
            jax.ShapeDtypeStruct((N, FW), jnp.float32),
            jax.ShapeDtypeStruct((N, D), jnp.float32),
        ],
    )(x, W, al16, ar16)


def _lane_bcast(vec, h):
    idx = jnp.full((D, 1), h, jnp.int32)
    dnums = lax.GatherDimensionNumbers(
        offset_dims=(), collapsed_slice_dims=(0,), start_index_map=(0,))
    return lax.gather(vec, idx, dnums, (1,),
                      mode=lax.GatherScatterMode.PROMISE_IN_BOUNDS)


def _sc_edge_body(src_hbm, dst_hbm, r_hbm, fa_hbm, fb_hbm,
                  accp_hbm,
                  srcv0, dstv0, rg0, fg0,
                  srcv1, dstv1, rg1, fg1,
                  sema0, sema1, semb0, semb1,
                  shared_acc):
    cid = lax.axis_index("c")
    sid = lax.axis_index("s")
    _Z16 = jnp.zeros((D,), jnp.float32)
    h0 = cid * HH
    ebase = sid * EPS

    bufs = ((srcv0, dstv0, rg0, fg0, sema0, semb0),
            (srcv1, dstv1, rg1, fg1, sema1, semb1))

    def _idx_issue(p, base):
        sv, dv, _, _, sa, _ = bufs[p]
        pltpu.async_copy(src_hbm.at[pl.ds(base, G)], sv, sa)
        pltpu.async_copy(dst_hbm.at[pl.ds(base, G)], dv, sa)

    def _idx_wait(p, base):
        sv, dv, _, _, sa, _ = bufs[p]
        pltpu.make_async_copy(src_hbm.at[pl.ds(base, G)], sv, sa).wait()
        pltpu.make_async_copy(dst_hbm.at[pl.ds(base, G)], dv, sa).wait()

    def _gather_issue(p):
        sv, dv, rgb, fgb, _, sb = bufs[p]
        pltpu.async_copy(r_hbm.at[dv], rgb, sb)

        @pl.when(cid == 0)
        def _():
            pltpu.async_copy(fa_hbm.at[sv], fgb, sb)

        @pl.when(cid == 1)
        def _():
            pltpu.async_copy(fb_hbm.at[sv], fgb, sb)

    def _gather_wait(p):
        sv, dv, rgb, fgb, _, sb = bufs[p]
        pltpu.make_async_copy(r_hbm.at[dv], rgb, sb).wait()

        @pl.when(cid == 0)
        def _():
            pltpu.make_async_copy(fa_hbm.at[sv], fgb, sb).wait()

        @pl.when(cid == 1)
        def _():
            pltpu.make_async_copy(fb_hbm.at[sv], fgb, sb).wait()

    def _compute_scatter(p):
        _, dv, rgb, fgb, _, _ = bufs[p]

        @plsc.parallel_loop(0, G, 1, unroll=8)
        def _(i):
            e = fgb[i, pl.ds(FH, D)] + rgb[i, :]
            e = jnp.where(e >= 0.0, e, e * jnp.float32(0.2))
            ex = jnp.exp(e)
            fgb[i, pl.ds(FH, D)] = ex
            for j in range(HH):
                m = _lane_bcast(ex, h0 + j)
                fgb[i, pl.ds(j * D, D)] = fgb[i, pl.ds(j * D, D)] * m

        pltpu.sync_copy(fgb, shared_acc.at[dv], add=True)

    _idx_issue(0, ebase)
    _idx_issue(1, ebase + G)

    @pl.loop(0, G)
    def _(r):
        for c in range(FW // D):
            fg0[r, pl.ds(c * D, D)] = _Z16

    row0 = sid * ROWS_PER_SUB
    done = 0
    while done + G <= ROWS_PER_SUB:
        pltpu.sync_copy(fg0, shared_acc.at[pl.ds(row0 + done, G)])
        done += G
    if ROWS_PER_SUB - done:
        pltpu.sync_copy(fg0.at[pl.ds(0, ROWS_PER_SUB - done)],
                        shared_acc.at[pl.ds(row0 + done,
                                            ROWS_PER_SUB - done)])

    @pl.when(sid == 0)
    def _():
        pltpu.sync_copy(fg0.at[pl.ds(0, TAIL)],
                        shared_acc.at[pl.ds(TAIL_ROW0, TAIL)])

    plsc.subcore_barrier()

    _idx_wait(0, ebase)
    _gather_issue(0)

    @pl.loop(0, NCHUNK - 2, step=2)
    def _(k):
        base = ebase + k * G
        _idx_wait(1, base + G)
        _gather_issue(1)
        _gather_wait(0)
        _compute_scatter(0)
        _idx_issue(0, base + 2 * G)
        _idx_wait(0, base + 2 * G)
        _gather_issue(0)
        _gather_wait(1)
        _compute_scatter(1)
        _idx_issue(1, base + 3 * G)

    _idx_wait(1, ebase + (NCHUNK - 1) * G)
    _gather_issue(1)
    _gather_wait(0)
    _compute_scatter(0)
    _gather_wait(1)
    _compute_scatter(1)

    plsc.subcore_barrier()
    pltpu.sync_copy(shared_acc.at[pl.ds(row0, ROWS_PER_SUB)],
                    accp_hbm.at[cid, pl.ds(row0, ROWS_PER_SUB)])

    @pl.when(sid == 0)
    def _():
        pltpu.sync_copy(shared_acc.at[pl.ds(TAIL_ROW0, TAIL)],
                        accp_hbm.at[cid, pl.ds(TAIL_ROW0, TAIL)])


def _sc_edges(src, dst, rtab, feat_a, feat_b):
    mesh = plsc.VectorSubcoreMesh(core_axis_name="c", subcore_axis_name="s")
    kern = pl.kernel(
        _sc_edge_body,
        mesh=mesh,
        compiler_params=pltpu.CompilerParams(use_tc_tiling_on_sc=False),
        out_type=[
            jax.ShapeDtypeStruct((NC, N, FW), jnp.float32),
        ],
        scratch_types=[
            pltpu.VMEM((G,), jnp.int32),
            pltpu.VMEM((G,), jnp.int32),
            pltpu.VMEM((G, D), jnp.float32),
            pltpu.VMEM((G, FW), jnp.float32),
            pltpu.VMEM((G,), jnp.int32),
            pltpu.VMEM((G,), jnp.int32),
            pltpu.VMEM((G, D), jnp.float32),
            pltpu.VMEM((G, FW), jnp.float32),
            pltpu.SemaphoreType.DMA,
            pltpu.SemaphoreType.DMA,
            pltpu.SemaphoreType.DMA,
            pltpu.SemaphoreType.DMA,
            pltpu.VMEM_SHARED((N, FW), jnp.float32),
        ],
    )
    return kern(src, dst, rtab, feat_a, feat_b)


def _epi_body(p_ref, x_ref, b_ref, o_ref):
    parts = []
    for h in range(H):
        c = h // HH
        j = h % HH
        dh = jnp.maximum(p_ref[c][:, FH + h:FH + h + 1], jnp.float32(1e-9))
        parts.append(p_ref[c][:, j * D:(j + 1) * D] / dh)
    r = jnp.concatenate(parts, axis=1)
    v = r + x_ref[...] + b_ref[...]
    o_ref[...] = jnp.where(v > 0.0, v, jnp.exp(jnp.minimum(v, 0.0)) - 1.0)


def _epilogue(accp, x, bias):
    blk = 1000
    return pl.pallas_call(
        _epi_body,
        grid=(N // blk,),
        in_specs=[
            pl.BlockSpec((NC, blk, FW), lambda i: (0, i, 0)),
            pl.BlockSpec((blk, F), lambda i: (i, 0)),
            pl.BlockSpec((1, F), lambda i: (0, 0)),
        ],
        out_specs=pl.BlockSpec((blk, F), lambda i: (i, 0)),
        out_shape=jax.ShapeDtypeStruct((N, F), jnp.float32),
    )(accp, x, bias.reshape(1, F))


@jax.jit
def kernel(adj, x, W, a_l, a_r, bias):
    src = adj[0].astype(jnp.int32)
    dst = adj[1].astype(jnp.int32)
    x = x.astype(jnp.float32)

    eye8 = jnp.eye(H, dtype=jnp.float32)
    al = (a_l[:, :, None] * eye8[:, None, :]).reshape(F, H)
    ar = (a_r[:, :, None] * eye8[:, None, :]).reshape(F, H)
    zpad = jnp.zeros((F, D - H), jnp.float32)
    al16 = jnp.concatenate([al, zpad], axis=1)
    ar16 = jnp.concatenate([ar, zpad], axis=1)

    feat_a, feat_b, rtab = _front(x, W, al16, ar16)
    (accp,) = _sc_edges(src, dst, rtab, feat_a, feat_b)
    return _epilogue(accp, x, bias.astype(jnp.float32))


# speedup vs baseline: 1.0680x; 1.0680x over previous
import jax
import jax.numpy as jnp
from jax import lax
from jax.experimental import pallas as pl
from jax.experimental.pallas import tpu as pltpu
from jax.experimental.pallas import tpu_sc as plsc

N = 10000
E = 320000
H = 8
D = 16
F = 128
HH = 4
FH = 64
FW = 80

NC = 2
NS = 16
EPS = E // NS
G = 200
NCHUNK = EPS // G

ROWS_PER_SUB = 624
TAIL_ROW0 = ROWS_PER_SUB * NS
TAIL = N - TAIL_ROW0


def _front_body(x_ref, w_ref, al_ref, ar_ref, fa_ref, fb_ref, r_ref):
    f = jnp.dot(x_ref[...], w_ref[...], preferred_element_type=jnp.float32)
    ltab = jnp.dot(f, al_ref[...], preferred_element_type=jnp.float32)
    fa_ref[...] = jnp.concatenate([f[:, :FH], ltab], axis=1)
    fb_ref[...] = jnp.concatenate([f[:, FH:], ltab], axis=1)
    r_ref[...] = jnp.dot(f, ar_ref[...], preferred_element_type=jnp.float32)


def _front(x, W, al16, ar16):
    blk = 1000
    return pl.pallas_call(
        _front_body,
        grid=(N // blk,),
        in_specs=[
            pl.BlockSpec((blk, F), lambda i: (i, 0)),
            pl.BlockSpec((F, F), lambda i: (0, 0)),
            pl.BlockSpec((F, D), lambda i: (0, 0)),
            pl.BlockSpec((F, D), lambda i: (0, 0)),
        ],
        out_specs=[
            pl.BlockSpec((blk, FW), lambda i: (i, 0)),
            pl.BlockSpec((blk, FW), lambda i: (i, 0)),
            pl.BlockSpec((blk, D), lambda i: (i, 0)),
        ],
        out_shape=[
            jax.ShapeDtypeStruct((N, FW), jnp.float32),
            jax.ShapeDtypeStruct((N, FW), jnp.float32),
            jax.ShapeDtypeStruct((N, D), jnp.float32),
        ],
    )(x, W, al16, ar16)


def _lane_bcast(vec, h):
    idx = jnp.full((D, 1), h, jnp.int32)
    dnums = lax.GatherDimensionNumbers(
        offset_dims=(), collapsed_slice_dims=(0,), start_index_map=(0,))
    return lax.gather(vec, idx, dnums, (1,),
                      mode=lax.GatherScatterMode.PROMISE_IN_BOUNDS)


def _sc_edge_body(src_hbm, dst_hbm, r_hbm, fa_hbm, fb_hbm,
                  accp_hbm,
                  srcv0, dstv0, rg0, fg0,
                  srcv1, dstv1, rg1, fg1,
                  srcv2, dstv2, rg2, fg2,
                  sema0, sema1, sema2, semb0, semb1, semb2,
                  semc0, semc1, semc2,
                  shared_acc):
    cid = lax.axis_index("c")
    sid = lax.axis_index("s")
    _Z16 = jnp.zeros((D,), jnp.float32)
    h0 = cid * HH
    ebase = sid * EPS

    bufs = ((srcv0, dstv0, rg0, fg0, sema0, semb0, semc0),
            (srcv1, dstv1, rg1, fg1, sema1, semb1, semc1),
            (srcv2, dstv2, rg2, fg2, sema2, semb2, semc2))

    def _idx_issue(p, base):
        sv, dv, _, _, sa, _, _ = bufs[p]
        pltpu.async_copy(src_hbm.at[pl.ds(base, G)], sv, sa)
        pltpu.async_copy(dst_hbm.at[pl.ds(base, G)], dv, sa)

    def _idx_wait(p, base):
        sv, dv, _, _, sa, _, _ = bufs[p]
        pltpu.make_async_copy(src_hbm.at[pl.ds(base, G)], sv, sa).wait()
        pltpu.make_async_copy(dst_hbm.at[pl.ds(base, G)], dv, sa).wait()

    def _gather_issue(p):
        sv, dv, rgb, fgb, _, sb, _ = bufs[p]
        pltpu.async_copy(r_hbm.at[dv], rgb, sb)

        @pl.when(cid == 0)
        def _():
            pltpu.async_copy(fa_hbm.at[sv], fgb, sb)

        @pl.when(cid == 1)
        def _():
            pltpu.async_copy(fb_hbm.at[sv], fgb, sb)

    def _gather_wait(p):
        sv, dv, rgb, fgb, _, sb, _ = bufs[p]
        pltpu.make_async_copy(r_hbm.at[dv], rgb, sb).wait()

        @pl.when(cid == 0)
        def _():
            pltpu.make_async_copy(fa_hbm.at[sv], fgb, sb).wait()

        @pl.when(cid == 1)
        def _():
            pltpu.make_async_copy(fb_hbm.at[sv], fgb, sb).wait()

    def _compute(p):
        _, _, rgb, fgb, _, _, _ = bufs[p]

        @plsc.parallel_loop(0, G, 1, unroll=4)
        def _(i):
            e = fgb[i, pl.ds(FH, D)] + rgb[i, :]
            e = jnp.where(e >= 0.0, e, e * jnp.float32(0.2))
            ex = jnp.exp(e)
            fgb[i, pl.ds(FH, D)] = ex
            for j in range(HH):
                m = _lane_bcast(ex, h0 + j)
                fgb[i, pl.ds(j * D, D)] = fgb[i, pl.ds(j * D, D)] * m

    def _scatter_issue(p):
        _, dv, _, fgb, _, _, sc = bufs[p]
        pltpu.async_copy(fgb, shared_acc.at[dv], sc, add=True)

    def _scatter_wait(p):
        _, dv, _, fgb, _, _, sc = bufs[p]
        pltpu.make_async_copy(fgb, shared_acc.at[dv], sc).wait()

    def _chunk(base, p, wait_prev_scatter=True, prefetch_gather=True,
               issue_idx2=True):
        p1 = (p + 1) % 3
        p2 = (p + 2) % 3
        if prefetch_gather:
            _idx_wait(p1, base + G)
            _gather_issue(p1)
        _gather_wait(p)
        _compute(p)
        if wait_prev_scatter:
            _scatter_wait(p2)
        _scatter_issue(p)
        if issue_idx2:
            _idx_issue(p2, base + 2 * G)

    _idx_issue(0, ebase)
    _idx_issue(1, ebase + G)

    @pl.loop(0, G)
    def _(r):
        for c in range(FW // D):
            fg0[r, pl.ds(c * D, D)] = _Z16

    row0 = sid * ROWS_PER_SUB
    done = 0
    while done + G <= ROWS_PER_SUB:
        pltpu.sync_copy(fg0, shared_acc.at[pl.ds(row0 + done, G)])
        done += G
    if ROWS_PER_SUB - done:
        pltpu.sync_copy(fg0.at[pl.ds(0, ROWS_PER_SUB - done)],
                        shared_acc.at[pl.ds(row0 + done,
                                            ROWS_PER_SUB - done)])

    @pl.when(sid == 0)
    def _():
        pltpu.sync_copy(fg0.at[pl.ds(0, TAIL)],
                        shared_acc.at[pl.ds(TAIL_ROW0, TAIL)])

    plsc.subcore_barrier()

    _idx_wait(0, ebase)
    _gather_issue(0)

    _chunk(ebase, 0, wait_prev_scatter=False)

    @pl.loop(1, NCHUNK - 3, step=3)
    def _(k):
        base = ebase + k * G
        _chunk(base, 1)
        _chunk(base + G, 2)
        _chunk(base + 2 * G, 0)

    base = ebase + (NCHUNK - 3) * G
    _chunk(base, 1)
    _chunk(base + G, 2, issue_idx2=False)
    _chunk(base + 2 * G, 0, prefetch_gather=False, issue_idx2=False)
    _scatter_wait(0)

    plsc.subcore_barrier()
    pltpu.sync_copy(shared_acc.at[pl.ds(row0, ROWS_PER_SUB)],
                    accp_hbm.at[cid, pl.ds(row0, ROWS_PER_SUB)])

    @pl.when(sid == 0)
    def _():
        pltpu.sync_copy(shared_acc.at[pl.ds(TAIL_ROW0, TAIL)],
                        accp_hbm.at[cid, pl.ds(TAIL_ROW0, TAIL)])


def _sc_edges(src, dst, rtab, feat_a, feat_b):
    mesh = plsc.VectorSubcoreMesh(core_axis_name="c", subcore_axis_name="s")
    kern = pl.kernel(
        _sc_edge_body,
        mesh=mesh,
        compiler_params=pltpu.CompilerParams(use_tc_tiling_on_sc=False),
        out_type=[
            jax.ShapeDtypeStruct((NC, N, FW), jnp.float32),
        ],
        scratch_types=(
            [pltpu.VMEM((G,), jnp.int32),
             pltpu.VMEM((G,), jnp.int32),
             pltpu.VMEM((G, D), jnp.float32),
             pltpu.VMEM((G, FW), jnp.float32)] * 3
            + [pltpu.SemaphoreType.DMA] * 9
            + [pltpu.VMEM_SHARED((N, FW), jnp.float32)]
        ),
    )
    return kern(src, dst, rtab, feat_a, feat_b)


def _epi_body(p_ref, x_ref, b_ref, o_ref):
    parts = []
    for h in range(H):
        c = h // HH
        j = h % HH
        dh = jnp.maximum(p_ref[c][:, FH + h:FH + h + 1], jnp.float32(1e-9))
        parts.append(p_ref[c][:, j * D:(j + 1) * D] / dh)
    r = jnp.concatenate(parts, axis=1)
    v = r + x_ref[...] + b_ref[...]
    o_ref[...] = jnp.where(v > 0.0, v, jnp.exp(jnp.minimum(v, 0.0)) - 1.0)


def _epilogue(accp, x, bias):
    blk = 1000
    return pl.pallas_call(
        _epi_body,
        grid=(N // blk,),
        in_specs=[
            pl.BlockSpec((NC, blk, FW), lambda i: (0, i, 0)),
            pl.BlockSpec((blk, F), lambda i: (i, 0)),
            pl.BlockSpec((1, F), lambda i: (0, 0)),
        ],
        out_specs=pl.BlockSpec((blk, F), lambda i: (i, 0)),
        out_shape=jax.ShapeDtypeStruct((N, F), jnp.float32),
    )(accp, x, bias.reshape(1, F))


@jax.jit
def kernel(adj, x, W, a_l, a_r, bias):
    src = adj[0].astype(jnp.int32)
    dst = adj[1].astype(jnp.int32)
    x = x.astype(jnp.float32)

    eye8 = jnp.eye(H, dtype=jnp.float32)
    al = (a_l[:, :, None] * eye8[:, None, :]).reshape(F, H)
    ar = (a_r[:, :, None] * eye8[:, None, :]).reshape(F, H)
    zpad = jnp.zeros((F, D - H), jnp.float32)
    al16 = jnp.concatenate([al, zpad], axis=1)
    ar16 = jnp.concatenate([ar, zpad], axis=1)

    feat_a, feat_b, rtab = _front(x, W, al16, ar16)
    (accp,) = _sc_edges(src, dst, rtab, feat_a, feat_b)
    return _epilogue(accp, x, bias.astype(jnp.float32))
